# baseline (speedup 1.0000x reference)
import jax
import jax.numpy as jnp
from jax.experimental import pallas as pl
from jax.experimental.pallas import tpu as pltpu

N = 4096
D_IN = 256
H = 128
C = 32
K = 4
LAM_HAT = 0.9
A_SCAD = 3.7
EPS = 1e-8

BI = 256
BJ = 512
BP = 256


def _prep_kernel(A_ref, F_ref, W1_ref, b1_ref, W2_ref, b2_ref,
                 F0_ref, Dd_ref, dinv_ref):
    a = A_ref[...]
    dd = jnp.sum(a, axis=1, keepdims=True) + 1.0
    Dd_ref[...] = dd
    dinv_ref[...] = jax.lax.rsqrt(dd)
    h = jnp.maximum(
        jnp.dot(F_ref[...], W1_ref[...], preferred_element_type=jnp.float32)
        + b1_ref[...], 0.0)
    F0_ref[...] = (jnp.dot(h, W2_ref[...], preferred_element_type=jnp.float32)
                   + b2_ref[...])


def _iter_kernel(lam_ref, A_ref, Fc_ref, dinv_ref, Dd_ref, F0_ref,
                 out_ref, P_acc):
    i = pl.program_id(0)
    j = pl.program_id(1)
    nj = pl.num_programs(1)

    lam_k = lam_ref[0]
    lam = 1.0 / LAM_HAT - 1.0
    alam = A_SCAD * lam_k
    inv_c = 1.0 / (2.0 * (A_SCAD - 1.0) * lam_k)

    dv_i = dinv_ref[pl.ds(i * BI, BI), :]
    xni = Fc_ref[pl.ds(i * BI, BI), :] * dv_i
    dv_j = dinv_ref[pl.ds(j * BJ, BJ), :]
    xnj = Fc_ref[pl.ds(j * BJ, BJ), :] * dv_j

    sqi = jnp.sum(xni * xni, axis=1, keepdims=True)
    sqj = jnp.sum(xnj * xnj, axis=1, keepdims=True)

    ones_i = jnp.ones((BI, 1), jnp.float32)
    ones_j = jnp.ones((BJ, 1), jnp.float32)
    ai = jnp.concatenate([xni * -2.0, sqi, ones_i], axis=1)
    bj = jnp.concatenate([xnj, ones_j, sqj], axis=1)
    zpre = jax.lax.dot_general(ai, bj, (((1,), (1,)), ((), ())),
                               preferred_element_type=jnp.float32)

    z = jnp.maximum(zpre, 0.0)
    r = jax.lax.rsqrt(jnp.maximum(z, EPS * EPS))
    y = z * r
    t = jnp.maximum(jnp.minimum(alam * inv_c - y * inv_c, 0.5), 0.0)
    w = t * r

    row = i * BI + jax.lax.broadcasted_iota(jnp.int32, (BI, BJ), 0)
    col = j * BJ + jax.lax.broadcasted_iota(jnp.int32, (BI, BJ), 1)
    w = jnp.where(row == col, 0.0, w)

    wa = w * A_ref[...]
    bj2 = jnp.concatenate([xnj, ones_j], axis=1)
    p_part = jax.lax.dot_general(wa, bj2, (((1,), (0,)), ((), ())),
                                 preferred_element_type=jnp.float32)

    @pl.when(j == 0)
    def _():
        P_acc[...] = p_part

    @pl.when(j > 0)
    def _():
        P_acc[...] += p_part

    @pl.when(j == nj - 1)
    def _():
        q = P_acc[:, C:C + 1] / Dd_ref[...] + lam
        out_ref[...] = (dv_i * P_acc[:, :C] + lam * F0_ref[...]) / q


def _prep_call(A, F, W1, b1, W2, b2):
    return pl.pallas_call(
        _prep_kernel,
        grid=(N // BP,),
        in_specs=[
            pl.BlockSpec((BP, N), lambda i: (i, 0)),
            pl.BlockSpec((BP, D_IN), lambda i: (i, 0)),
            pl.BlockSpec((D_IN, H), lambda i: (0, 0)),
            pl.BlockSpec((1, H), lambda i: (0, 0)),
            pl.BlockSpec((H, C), lambda i: (0, 0)),
            pl.BlockSpec((1, C), lambda i: (0, 0)),
        ],
        out_specs=[
            pl.BlockSpec((BP, C), lambda i: (i, 0)),
            pl.BlockSpec((BP, 1), lambda i: (i, 0)),
            pl.BlockSpec((BP, 1), lambda i: (i, 0)),
        ],
        out_shape=[
            jax.ShapeDtypeStruct((N, C), jnp.float32),
            jax.ShapeDtypeStruct((N, 1), jnp.float32),
            jax.ShapeDtypeStruct((N, 1), jnp.float32),
        ],
        compiler_params=pltpu.CompilerParams(
            dimension_semantics=("arbitrary",)),
    )(A, F, W1, b1, W2, b2)


def _iter_call(lam_k, A, Fc, dinv, Dd, F0):
    return pl.pallas_call(
        _iter_kernel,
        grid=(N // BI, N // BJ),
        in_specs=[
            pl.BlockSpec(memory_space=pltpu.SMEM),
            pl.BlockSpec((BI, BJ), lambda i, j: (i, j)),
            pl.BlockSpec((N, C), lambda i, j: (0, 0)),
            pl.BlockSpec((N, 1), lambda i, j: (0, 0)),
            pl.BlockSpec((BI, 1), lambda i, j: (i, 0)),
            pl.BlockSpec((BI, C), lambda i, j: (i, 0)),
        ],
        out_specs=pl.BlockSpec((BI, C), lambda i, j: (i, 0)),
        out_shape=jax.ShapeDtypeStruct((N, C), jnp.float32),
        scratch_shapes=[
            pltpu.VMEM((BI, C + 1), jnp.float32),
        ],
        compiler_params=pltpu.CompilerParams(
            dimension_semantics=("parallel", "arbitrary")),
    )(lam_k, A, Fc, dinv, Dd, F0)


def kernel(A, F, W1, b1, W2, b2, log_lams):
    F0, Dd, dinv = _prep_call(A, F, W1, b1.reshape(1, H), W2, b2.reshape(1, C))
    lams = jnp.exp(log_lams)
    Fc = F0
    for k in range(K):
        Fc = _iter_call(lams[k].reshape(1), A, Fc, dinv, Dd, F0)
    return Fc


# speedup vs baseline: 1.8903x; 1.8903x over previous
import jax
import jax.numpy as jnp
import numpy as np
from jax.experimental import pallas as pl
from jax.experimental.pallas import tpu as pltpu

N = 4096
D_IN = 256
H = 128
C = 32
K = 4
LAM_HAT = 0.9
A_SCAD = 3.7
EPS = 1e-8

BT = 512
NT = N // BT
NPAIRS = NT * (NT + 1) // 2
BP = 256


def _prep_kernel(A_ref, F_ref, W1_ref, b1_ref, W2_ref, b2_ref,
                 F0_ref, Dd_ref, dinv_ref):
    a = A_ref[...]
    dd = jnp.sum(a, axis=1, keepdims=True) + 1.0
    Dd_ref[...] = dd
    dinv_ref[...] = jax.lax.rsqrt(dd)
    h = jnp.maximum(
        jnp.dot(F_ref[...], W1_ref[...], preferred_element_type=jnp.float32)
        + b1_ref[...], 0.0)
    F0_ref[...] = (jnp.dot(h, W2_ref[...], preferred_element_type=jnp.float32)
                   + b2_ref[...])


def _iter_kernel(ti_ref, tj_ref, lam_ref, A1_ref, A2_ref, Fc_ref, dinv_ref,
                 Dd_ref, F0_ref, out_ref, S_acc, P_acc):
    p = pl.program_id(0)
    ti = ti_ref[p]
    tj = tj_ref[p]
    lam_k = lam_ref[0]
    lam = 1.0 / LAM_HAT - 1.0
    alam = A_SCAD * lam_k
    inv_c = 1.0 / (2.0 * (A_SCAD - 1.0) * lam_k)

    @pl.when(p == 0)
    def _():
        S_acc[...] = jnp.zeros_like(S_acc)
        P_acc[...] = jnp.zeros_like(P_acc)

    @pl.when(p < NPAIRS)
    def _():
        xni = Fc_ref[pl.ds(ti * BT, BT), :] * dinv_ref[pl.ds(ti * BT, BT), :]
        xnj = Fc_ref[pl.ds(tj * BT, BT), :] * dinv_ref[pl.ds(tj * BT, BT), :]
        sqi = jnp.sum(xni * xni, axis=1, keepdims=True)
        sqj = jnp.sum(xnj * xnj, axis=1, keepdims=True).T

        g = jax.lax.dot_general(xni, xnj, (((1,), (1,)), ((), ())),
                                preferred_element_type=jnp.float32)
        z = jnp.maximum(sqi + sqj - 2.0 * g, 0.0)
        r = jax.lax.rsqrt(jnp.maximum(z, EPS * EPS))
        y = z * r
        t = jnp.maximum(jnp.minimum(alam * inv_c - y * inv_c, 0.5), 0.0)
        w = t * r
        row = jax.lax.broadcasted_iota(jnp.int32, (BT, BT), 0)
        col = jax.lax.broadcasted_iota(jnp.int32, (BT, BT), 1)
        w = jnp.where(jnp.logical_and(ti == tj, row == col), 0.0, w)

        wa1 = w * A1_ref[...]
        S_acc[pl.ds(ti * BT, BT), :] += jnp.sum(wa1, axis=1, keepdims=True)
        P_acc[pl.ds(ti * BT, BT), :] += jax.lax.dot_general(
            wa1, xnj, (((1,), (0,)), ((), ())),
            preferred_element_type=jnp.float32)

        @pl.when(ti != tj)
        def _():
            wt = w.T
            wa2 = wt * A2_ref[...]
            S_acc[pl.ds(tj * BT, BT), :] += jnp.sum(wa2, axis=1, keepdims=True)
            P_acc[pl.ds(tj * BT, BT), :] += jax.lax.dot_general(
                wa2, xni, (((1,), (0,)), ((), ())),
                preferred_element_type=jnp.float32)

    @pl.when(p == NPAIRS)
    def _():
        q = S_acc[...] / Dd_ref[...] + lam
        out_ref[...] = (dinv_ref[...] * P_acc[...] + lam * F0_ref[...]) / q


def _prep_call(A, F, W1, b1, W2, b2):
    return pl.pallas_call(
        _prep_kernel,
        grid=(N // BP,),
        in_specs=[
            pl.BlockSpec((BP, N), lambda i: (i, 0)),
            pl.BlockSpec((BP, D_IN), lambda i: (i, 0)),
            pl.BlockSpec((D_IN, H), lambda i: (0, 0)),
            pl.BlockSpec((1, H), lambda i: (0, 0)),
            pl.BlockSpec((H, C), lambda i: (0, 0)),
            pl.BlockSpec((1, C), lambda i: (0, 0)),
        ],
        out_specs=[
            pl.BlockSpec((BP, C), lambda i: (i, 0)),
            pl.BlockSpec((BP, 1), lambda i: (i, 0)),
            pl.BlockSpec((BP, 1), lambda i: (i, 0)),
        ],
        out_shape=[
            jax.ShapeDtypeStruct((N, C), jnp.float32),
            jax.ShapeDtypeStruct((N, 1), jnp.float32),
            jax.ShapeDtypeStruct((N, 1), jnp.float32),
        ],
        compiler_params=pltpu.CompilerParams(
            dimension_semantics=("arbitrary",)),
    )(A, F, W1, b1, W2, b2)


_TI_LIST = []
_TJ_LIST = []
for _a in range(NT):
    for _b in range(_a, NT):
        _TI_LIST.append(_a)
        _TJ_LIST.append(_b)
_TI_LIST.append(0)
_TJ_LIST.append(0)
_TI_ARR = np.asarray(_TI_LIST, np.int32)
_TJ_ARR = np.asarray(_TJ_LIST, np.int32)


def _iter_call(lam_k, A, Fc, dinv, Dd, F0):
    grid_spec = pltpu.PrefetchScalarGridSpec(
        num_scalar_prefetch=3,
        grid=(NPAIRS + 1,),
        in_specs=[
            pl.BlockSpec((BT, BT), lambda p, ti, tj, lam: (ti[p], tj[p])),
            pl.BlockSpec((BT, BT), lambda p, ti, tj, lam: (tj[p], ti[p])),
            pl.BlockSpec((N, C), lambda p, ti, tj, lam: (0, 0)),
            pl.BlockSpec((N, 1), lambda p, ti, tj, lam: (0, 0)),
            pl.BlockSpec((N, 1), lambda p, ti, tj, lam: (0, 0)),
            pl.BlockSpec((N, C), lambda p, ti, tj, lam: (0, 0)),
        ],
        out_specs=pl.BlockSpec((N, C), lambda p, ti, tj, lam: (0, 0)),
        scratch_shapes=[
            pltpu.VMEM((N, 1), jnp.float32),
            pltpu.VMEM((N, C), jnp.float32),
        ],
    )
    return pl.pallas_call(
        _iter_kernel,
        grid_spec=grid_spec,
        out_shape=jax.ShapeDtypeStruct((N, C), jnp.float32),
        compiler_params=pltpu.CompilerParams(
            dimension_semantics=("arbitrary",)),
    )(jnp.asarray(_TI_ARR), jnp.asarray(_TJ_ARR), lam_k,
      A, A, Fc, dinv, Dd, F0)


def kernel(A, F, W1, b1, W2, b2, log_lams):
    F0, Dd, dinv = _prep_call(A, F, W1, b1.reshape(1, H), W2, b2.reshape(1, C))
    lams = jnp.exp(log_lams)
    Fc = F0
    for k in range(K):
        Fc = _iter_call(lams[k].reshape(1), A, Fc, dinv, Dd, F0)
    return Fc
